# baseline (speedup 1.0000x reference)
import functools

import jax
import jax.numpy as jnp
from jax import lax
from jax.experimental import pallas as pl
from jax.experimental.pallas import tpu as pltpu
from jax.experimental.pallas import tpu_sc as plsc

N = 10000
D = 128
NC = 2
NT = 16
CH = 64
K = 160
KG = 2
G = K // KG
EPAD = NC * NT * K * CH
NPAD = 10240
RPT = NPAD // NT
ZCH = 64
BLK = 640
BLK2 = 400


def _mesh():
    return plsc.VectorSubcoreMesh(core_axis_name="c", subcore_axis_name="s")


def _zero_acc(zrow, buf, acc_s, base):
    del buf
    for t in range(RPT // ZCH):
        pltpu.sync_copy(zrow, acc_s.at[pl.ds(base + t * ZCH, ZCH)])


def _stage_out(acc_s, buf, out_hbm, cid, base):
    del buf
    for t in range(RPT // ZCH):
        sl = pl.ds(base + t * ZCH, ZCH)
        pltpu.sync_copy(acc_s.at[sl], out_hbm.at[cid, sl])


def _agg_loop(x_hbm, src_hbm, dst_hbm, cid, sid,
              src_v, dst_v, bufa, bufb, acc_s, semga, semgb):
    def group(g, carry):
        pltpu.sync_copy(src_hbm.at[cid, sid, g], src_v)
        pltpu.sync_copy(dst_hbm.at[cid, sid, g], dst_v)
        pltpu.async_copy(x_hbm.at[src_v.at[0]], bufa, semga)

        def pair(i, c2):
            j = 2 * i
            pltpu.async_copy(x_hbm.at[src_v.at[j + 1]], bufb, semgb)
            pltpu.make_async_copy(x_hbm.at[src_v.at[j]], bufa, semga).wait()
            pltpu.sync_copy(bufa, acc_s.at[dst_v.at[j]], add=True)
            pltpu.async_copy(x_hbm.at[src_v.at[j + 2]], bufa, semga)
            pltpu.make_async_copy(x_hbm.at[src_v.at[j + 1]], bufb,
                                  semgb).wait()
            pltpu.sync_copy(bufb, acc_s.at[dst_v.at[j + 1]], add=True)
            return c2

        lax.fori_loop(0, G // 2, pair, 0)
        pltpu.make_async_copy(x_hbm.at[src_v.at[G]], bufa, semga).wait()
        return carry

    lax.fori_loop(0, KG, group, 0)


def _count_loop(dst_hbm, cid, sid, dst_v, buf, acc_s):
    def group(g, carry):
        pltpu.sync_copy(dst_hbm.at[cid, sid, g], dst_v)

        def step(j, c2):
            pltpu.sync_copy(buf, acc_s.at[dst_v.at[j]], add=True)
            return c2

        lax.fori_loop(0, G, step, 0)
        return carry

    lax.fori_loop(0, KG, group, 0)


def _sc_cnt_agg_body(x_hbm, src_hbm, dst_hbm, zrow, ones,
                     cnt_out, acc_out,
                     src_v, dst_v, bufa, bufb, acc_s,
                     semga, semgb):
    cid = lax.axis_index("c")
    sid = lax.axis_index("s")
    base = sid * RPT

    _zero_acc(zrow, bufa, acc_s, base)
    pltpu.sync_copy(ones, bufa)
    plsc.subcore_barrier()
    _count_loop(dst_hbm, cid, sid, dst_v, bufa, acc_s)
    plsc.subcore_barrier()
    _stage_out(acc_s, bufa, cnt_out, cid, base)

    _zero_acc(zrow, bufa, acc_s, base)
    plsc.subcore_barrier()
    _agg_loop(x_hbm, src_hbm, dst_hbm, cid, sid,
              src_v, dst_v, bufa, bufb, acc_s, semga, semgb)
    plsc.subcore_barrier()
    _stage_out(acc_s, bufa, acc_out, cid, base)


_sc_cnt_agg = pl.kernel(
    _sc_cnt_agg_body,
    out_type=[jax.ShapeDtypeStruct((NC, NPAD, D), jnp.float32),
              jax.ShapeDtypeStruct((NC, NPAD, D), jnp.float32)],
    mesh=_mesh(),
    scratch_types=[
        pltpu.VMEM((G + 2, CH), jnp.int32),
        pltpu.VMEM((G, CH), jnp.int32),
        pltpu.VMEM((CH, D), jnp.float32),
        pltpu.VMEM((CH, D), jnp.float32),
        pltpu.VMEM_SHARED((NPAD, D), jnp.float32),
        pltpu.SemaphoreType.DMA,
        pltpu.SemaphoreType.DMA,
    ],
)


def _sc_agg_body(x_hbm, src_hbm, dst_hbm, zrow, acc_out,
                 src_v, dst_v, bufa, bufb, acc_s,
                 semga, semgb):
    cid = lax.axis_index("c")
    sid = lax.axis_index("s")
    base = sid * RPT

    _zero_acc(zrow, bufa, acc_s, base)
    plsc.subcore_barrier()
    _agg_loop(x_hbm, src_hbm, dst_hbm, cid, sid,
              src_v, dst_v, bufa, bufb, acc_s, semga, semgb)
    plsc.subcore_barrier()
    _stage_out(acc_s, bufa, acc_out, cid, base)


_sc_agg = pl.kernel(
    _sc_agg_body,
    out_type=jax.ShapeDtypeStruct((NC, NPAD, D), jnp.float32),
    mesh=_mesh(),
    scratch_types=[
        pltpu.VMEM((G + 2, CH), jnp.int32),
        pltpu.VMEM((G, CH), jnp.int32),
        pltpu.VMEM((CH, D), jnp.float32),
        pltpu.VMEM((CH, D), jnp.float32),
        pltpu.VMEM_SHARED((NPAD, D), jnp.float32),
        pltpu.SemaphoreType.DMA,
        pltpu.SemaphoreType.DMA,
    ],
)


def _dot_t(a, w):
    return lax.dot_general(a, w, (((1,), (1,)), ((), ())),
                           preferred_element_type=jnp.float32)


def _tc_layer_body(normalize, acc_ref, cnt_ref, x_ref, wl_ref, wr_ref, b_ref,
                   o_ref):
    agg = acc_ref[0] + acc_ref[1]
    cnt = cnt_ref[0, :, :1] + cnt_ref[1, :, :1]
    a = agg / jnp.maximum(cnt, 1.0)
    h = _dot_t(a, wl_ref[...]) + _dot_t(x_ref[...], wr_ref[...]) + b_ref[...]
    if normalize:
        nrm = jnp.sqrt(jnp.sum(h * h, axis=1, keepdims=True))
        h = h / jnp.maximum(nrm, 1e-12)
    o_ref[...] = jax.nn.sigmoid(h)


def _make_tc_layer(normalize):
    return pl.pallas_call(
        functools.partial(_tc_layer_body, normalize),
        grid=(N // BLK2,),
        in_specs=[
            pl.BlockSpec((NC, BLK2, D), lambda i: (0, i, 0)),
            pl.BlockSpec((NC, BLK2, D), lambda i: (0, i, 0)),
            pl.BlockSpec((BLK2, D), lambda i: (i, 0)),
            pl.BlockSpec((D, D), lambda i: (0, 0)),
            pl.BlockSpec((D, D), lambda i: (0, 0)),
            pl.BlockSpec((1, D), lambda i: (0, 0)),
        ],
        out_specs=pl.BlockSpec((BLK2, D), lambda i: (i, 0)),
        out_shape=jax.ShapeDtypeStruct((N, D), jnp.float32),
    )


_tc_layer1 = _make_tc_layer(False)
_tc_layer2 = _make_tc_layer(True)


@jax.jit
def kernel(node_feat, edge_index, W1l, b1, W1r, W2l, b2, W2r):
    pad = EPAD - edge_index.shape[1]
    arange_pad = jnp.arange(pad, dtype=jnp.int32) % NT
    src_p = jnp.concatenate([edge_index[0], arange_pad]).reshape(
        NC, NT, KG, G, CH)
    dst_p = jnp.concatenate([edge_index[1], N + arange_pad]).reshape(
        NC, NT, KG, G, CH)
    xtra = jnp.broadcast_to(
        (jnp.arange(2 * CH, dtype=jnp.int32) % NT).reshape(1, 1, 1, 2, CH),
        (NC, NT, KG, 2, CH))
    src_p = jnp.concatenate([src_p, xtra], axis=3)

    zrow = jnp.zeros((ZCH, D), jnp.float32)
    ones = jnp.ones((ZCH, D), jnp.float32)

    cnt, acc1 = _sc_cnt_agg(node_feat, src_p, dst_p, zrow, ones)
    x1 = _tc_layer1(acc1, cnt, node_feat, W1l, W1r, b1.reshape(1, D))
    acc2 = _sc_agg(x1, src_p, dst_p, zrow)
    x2 = _tc_layer2(acc2, cnt, x1, W2l, W2r, b2.reshape(1, D))
    return x2


# speedup vs baseline: 1.1291x; 1.1291x over previous
import functools

import jax
import jax.numpy as jnp
from jax import lax
from jax.experimental import pallas as pl
from jax.experimental.pallas import tpu as pltpu
from jax.experimental.pallas import tpu_sc as plsc

N = 10000
D = 128
NC = 2
NT = 16
CH = 64
K = 160
KG = 2
G = K // KG
EPAD = NC * NT * K * CH
NPAD = 10240
RPT = NPAD // NT
ZCH = 64
BLK = 640
BLK2 = 400


def _mesh():
    return plsc.VectorSubcoreMesh(core_axis_name="c", subcore_axis_name="s")


def _zero_acc(zrow, buf, acc_s, base):
    pltpu.sync_copy(zrow, buf)
    for t in range(RPT // ZCH):
        pltpu.sync_copy(buf, acc_s.at[pl.ds(base + t * ZCH, ZCH)])


def _stage_out(acc_s, buf, out_hbm, cid, base):
    for t in range(RPT // ZCH):
        sl = pl.ds(base + t * ZCH, ZCH)
        pltpu.sync_copy(acc_s.at[sl], buf)
        pltpu.sync_copy(buf, out_hbm.at[cid, sl])


def _agg_loop(x_hbm, src_hbm, dst_hbm, cid, sid,
              src_v, dst_v, bufa, bufb, acc_s, semga, semgb):
    def group(g, carry):
        pltpu.sync_copy(src_hbm.at[cid, sid, g], src_v)
        pltpu.sync_copy(dst_hbm.at[cid, sid, g], dst_v)
        pltpu.async_copy(x_hbm.at[src_v.at[0]], bufa, semga)

        def pair(i, c2):
            j = 2 * i
            pltpu.async_copy(x_hbm.at[src_v.at[j + 1]], bufb, semgb)
            pltpu.make_async_copy(x_hbm.at[src_v.at[j]], bufa, semga).wait()
            pltpu.sync_copy(bufa, acc_s.at[dst_v.at[j]], add=True)
            pltpu.async_copy(x_hbm.at[src_v.at[j + 2]], bufa, semga)
            pltpu.make_async_copy(x_hbm.at[src_v.at[j + 1]], bufb,
                                  semgb).wait()
            pltpu.sync_copy(bufb, acc_s.at[dst_v.at[j + 1]], add=True)
            return c2

        lax.fori_loop(0, G // 2, pair, 0)
        pltpu.make_async_copy(x_hbm.at[src_v.at[G]], bufa, semga).wait()
        return carry

    lax.fori_loop(0, KG, group, 0)


def _count_loop(dst_hbm, cid, sid, dst_v, buf, acc_s):
    def group(g, carry):
        pltpu.sync_copy(dst_hbm.at[cid, sid, g], dst_v)

        def step(j, c2):
            pltpu.sync_copy(buf, acc_s.at[dst_v.at[j]], add=True)
            return c2

        lax.fori_loop(0, G, step, 0)
        return carry

    lax.fori_loop(0, KG, group, 0)


def _sc_cnt_agg_body(x_hbm, src_hbm, dst_hbm, zrow, ones,
                     cnt_out, acc_out,
                     src_v, dst_v, bufa, bufb, acc_s,
                     semga, semgb):
    cid = lax.axis_index("c")
    sid = lax.axis_index("s")
    base = sid * RPT

    _zero_acc(zrow, bufa, acc_s, base)
    pltpu.sync_copy(ones, bufa)
    plsc.subcore_barrier()
    _count_loop(dst_hbm, cid, sid, dst_v, bufa, acc_s)
    plsc.subcore_barrier()
    _stage_out(acc_s, bufa, cnt_out, cid, base)

    _zero_acc(zrow, bufa, acc_s, base)
    plsc.subcore_barrier()
    _agg_loop(x_hbm, src_hbm, dst_hbm, cid, sid,
              src_v, dst_v, bufa, bufb, acc_s, semga, semgb)
    plsc.subcore_barrier()
    _stage_out(acc_s, bufa, acc_out, cid, base)


_sc_cnt_agg = pl.kernel(
    _sc_cnt_agg_body,
    out_type=[jax.ShapeDtypeStruct((NC, NPAD, D), jnp.float32),
              jax.ShapeDtypeStruct((NC, NPAD, D), jnp.float32)],
    mesh=_mesh(),
    scratch_types=[
        pltpu.VMEM((G + 2, CH), jnp.int32),
        pltpu.VMEM((G, CH), jnp.int32),
        pltpu.VMEM((CH, D), jnp.float32),
        pltpu.VMEM((CH, D), jnp.float32),
        pltpu.VMEM_SHARED((NPAD, D), jnp.float32),
        pltpu.SemaphoreType.DMA,
        pltpu.SemaphoreType.DMA,
    ],
)


def _sc_agg_body(x_hbm, src_hbm, dst_hbm, zrow, acc_out,
                 src_v, dst_v, bufa, bufb, acc_s,
                 semga, semgb):
    cid = lax.axis_index("c")
    sid = lax.axis_index("s")
    base = sid * RPT

    _zero_acc(zrow, bufa, acc_s, base)
    plsc.subcore_barrier()
    _agg_loop(x_hbm, src_hbm, dst_hbm, cid, sid,
              src_v, dst_v, bufa, bufb, acc_s, semga, semgb)
    plsc.subcore_barrier()
    _stage_out(acc_s, bufa, acc_out, cid, base)


_sc_agg = pl.kernel(
    _sc_agg_body,
    out_type=jax.ShapeDtypeStruct((NC, NPAD, D), jnp.float32),
    mesh=_mesh(),
    scratch_types=[
        pltpu.VMEM((G + 2, CH), jnp.int32),
        pltpu.VMEM((G, CH), jnp.int32),
        pltpu.VMEM((CH, D), jnp.float32),
        pltpu.VMEM((CH, D), jnp.float32),
        pltpu.VMEM_SHARED((NPAD, D), jnp.float32),
        pltpu.SemaphoreType.DMA,
        pltpu.SemaphoreType.DMA,
    ],
)


def _dot_t(a, w):
    return lax.dot_general(a, w, (((1,), (1,)), ((), ())),
                           preferred_element_type=jnp.float32)


def _tc_layer_body(normalize, acc_ref, cnt_ref, x_ref, wl_ref, wr_ref, b_ref,
                   o_ref):
    agg = acc_ref[0] + acc_ref[1]
    cnt = cnt_ref[0, :, :1] + cnt_ref[1, :, :1]
    a = agg / jnp.maximum(cnt, 1.0)
    h = _dot_t(a, wl_ref[...]) + _dot_t(x_ref[...], wr_ref[...]) + b_ref[...]
    if normalize:
        nrm = jnp.sqrt(jnp.sum(h * h, axis=1, keepdims=True))
        h = h / jnp.maximum(nrm, 1e-12)
    o_ref[...] = jax.nn.sigmoid(h)


def _make_tc_layer(normalize):
    return pl.pallas_call(
        functools.partial(_tc_layer_body, normalize),
        grid=(N // BLK2,),
        in_specs=[
            pl.BlockSpec((NC, BLK2, D), lambda i: (0, i, 0)),
            pl.BlockSpec((NC, BLK2, D), lambda i: (0, i, 0)),
            pl.BlockSpec((BLK2, D), lambda i: (i, 0)),
            pl.BlockSpec((D, D), lambda i: (0, 0)),
            pl.BlockSpec((D, D), lambda i: (0, 0)),
            pl.BlockSpec((1, D), lambda i: (0, 0)),
        ],
        out_specs=pl.BlockSpec((BLK2, D), lambda i: (i, 0)),
        out_shape=jax.ShapeDtypeStruct((N, D), jnp.float32),
    )


_tc_layer1 = _make_tc_layer(False)
_tc_layer2 = _make_tc_layer(True)


@jax.jit
def kernel(node_feat, edge_index, W1l, b1, W1r, W2l, b2, W2r):
    pad = EPAD - edge_index.shape[1]
    arange_pad = jnp.arange(pad, dtype=jnp.int32) % NT
    src_p = jnp.concatenate([edge_index[0], arange_pad]).reshape(
        NC, NT, KG, G, CH)
    dst_p = jnp.concatenate([edge_index[1], N + arange_pad]).reshape(
        NC, NT, KG, G, CH)
    xtra = jnp.broadcast_to(
        (jnp.arange(2 * CH, dtype=jnp.int32) % NT).reshape(1, 1, 1, 2, CH),
        (NC, NT, KG, 2, CH))
    src_p = jnp.concatenate([src_p, xtra], axis=3)

    zrow = jnp.zeros((ZCH, D), jnp.float32)
    ones = jnp.ones((ZCH, D), jnp.float32)

    cnt, acc1 = _sc_cnt_agg(node_feat, src_p, dst_p, zrow, ones)
    x1 = _tc_layer1(acc1, cnt, node_feat, W1l, W1r, b1.reshape(1, D))
    acc2 = _sc_agg(x1, src_p, dst_p, zrow)
    x2 = _tc_layer2(acc2, cnt, x1, W2l, W2r, b2.reshape(1, D))
    return x2


# speedup vs baseline: 1.1756x; 1.0413x over previous
import functools

import jax
import jax.numpy as jnp
from jax import lax
from jax.experimental import pallas as pl
from jax.experimental.pallas import tpu as pltpu
from jax.experimental.pallas import tpu_sc as plsc

N = 10000
D = 128
NC = 2
NT = 16
CH = 64
K = 160
KG = 2
G = K // KG
EPAD = NC * NT * K * CH
NPAD = 10240
RPT = NPAD // NT
ZCH = 64
BLK = 640
BLK2 = 1000


def _mesh():
    return plsc.VectorSubcoreMesh(core_axis_name="c", subcore_axis_name="s")


def _zero_acc(zrow, buf, acc_s, base):
    pltpu.sync_copy(zrow, buf)
    for t in range(RPT // ZCH):
        pltpu.sync_copy(buf, acc_s.at[pl.ds(base + t * ZCH, ZCH)])


def _stage_out(acc_s, buf, out_hbm, cid, base):
    for t in range(RPT // ZCH):
        sl = pl.ds(base + t * ZCH, ZCH)
        pltpu.sync_copy(acc_s.at[sl], buf)
        pltpu.sync_copy(buf, out_hbm.at[cid, sl])


def _agg_loop(x_hbm, src_hbm, dst_hbm, cid, sid,
              src_v, dst_v, bufa, bufb, acc_s, semga, semgb):
    def group(g, carry):
        pltpu.sync_copy(src_hbm.at[cid, sid, g], src_v)
        pltpu.sync_copy(dst_hbm.at[cid, sid, g], dst_v)
        pltpu.async_copy(x_hbm.at[src_v.at[0]], bufa, semga)

        def pair(i, c2):
            j = 2 * i
            pltpu.async_copy(x_hbm.at[src_v.at[j + 1]], bufb, semgb)
            pltpu.make_async_copy(x_hbm.at[src_v.at[j]], bufa, semga).wait()
            pltpu.sync_copy(bufa, acc_s.at[dst_v.at[j]], add=True)
            pltpu.async_copy(x_hbm.at[src_v.at[j + 2]], bufa, semga)
            pltpu.make_async_copy(x_hbm.at[src_v.at[j + 1]], bufb,
                                  semgb).wait()
            pltpu.sync_copy(bufb, acc_s.at[dst_v.at[j + 1]], add=True)
            return c2

        lax.fori_loop(0, G // 2, pair, 0)
        pltpu.make_async_copy(x_hbm.at[src_v.at[G]], bufa, semga).wait()
        return carry

    lax.fori_loop(0, KG, group, 0)


def _count_loop(dst_hbm, cid, sid, dst_v, buf, acc_s):
    def group(g, carry):
        pltpu.sync_copy(dst_hbm.at[cid, sid, g], dst_v)

        def step(j, c2):
            pltpu.sync_copy(buf, acc_s.at[dst_v.at[j]], add=True)
            return c2

        lax.fori_loop(0, G, step, 0)
        return carry

    lax.fori_loop(0, KG, group, 0)


def _sc_cnt_agg_body(x_hbm, src_hbm, dst_hbm, zrow, ones,
                     cnt_out, acc_out,
                     src_v, dst_v, bufa, bufb, acc_s,
                     semga, semgb):
    cid = lax.axis_index("c")
    sid = lax.axis_index("s")
    base = sid * RPT

    _zero_acc(zrow, bufa, acc_s, base)
    pltpu.sync_copy(ones, bufa)
    plsc.subcore_barrier()
    _count_loop(dst_hbm, cid, sid, dst_v, bufa, acc_s)
    plsc.subcore_barrier()
    _stage_out(acc_s, bufa, cnt_out, cid, base)

    _zero_acc(zrow, bufa, acc_s, base)
    plsc.subcore_barrier()
    _agg_loop(x_hbm, src_hbm, dst_hbm, cid, sid,
              src_v, dst_v, bufa, bufb, acc_s, semga, semgb)
    plsc.subcore_barrier()
    _stage_out(acc_s, bufa, acc_out, cid, base)


_sc_cnt_agg = pl.kernel(
    _sc_cnt_agg_body,
    out_type=[jax.ShapeDtypeStruct((NC, NPAD, D), jnp.float32),
              jax.ShapeDtypeStruct((NC, NPAD, D), jnp.float32)],
    mesh=_mesh(),
    scratch_types=[
        pltpu.VMEM((G + 2, CH), jnp.int32),
        pltpu.VMEM((G, CH), jnp.int32),
        pltpu.VMEM((CH, D), jnp.float32),
        pltpu.VMEM((CH, D), jnp.float32),
        pltpu.VMEM_SHARED((NPAD, D), jnp.float32),
        pltpu.SemaphoreType.DMA,
        pltpu.SemaphoreType.DMA,
    ],
)


def _sc_agg_body(x_hbm, src_hbm, dst_hbm, zrow, acc_out,
                 src_v, dst_v, bufa, bufb, acc_s,
                 semga, semgb):
    cid = lax.axis_index("c")
    sid = lax.axis_index("s")
    base = sid * RPT

    _zero_acc(zrow, bufa, acc_s, base)
    plsc.subcore_barrier()
    _agg_loop(x_hbm, src_hbm, dst_hbm, cid, sid,
              src_v, dst_v, bufa, bufb, acc_s, semga, semgb)
    plsc.subcore_barrier()
    _stage_out(acc_s, bufa, acc_out, cid, base)


_sc_agg = pl.kernel(
    _sc_agg_body,
    out_type=jax.ShapeDtypeStruct((NC, NPAD, D), jnp.float32),
    mesh=_mesh(),
    scratch_types=[
        pltpu.VMEM((G + 2, CH), jnp.int32),
        pltpu.VMEM((G, CH), jnp.int32),
        pltpu.VMEM((CH, D), jnp.float32),
        pltpu.VMEM((CH, D), jnp.float32),
        pltpu.VMEM_SHARED((NPAD, D), jnp.float32),
        pltpu.SemaphoreType.DMA,
        pltpu.SemaphoreType.DMA,
    ],
)


def _dot_t(a, w):
    return lax.dot_general(a, w, (((1,), (1,)), ((), ())),
                           preferred_element_type=jnp.float32)


def _tc_layer_body(normalize, acc_ref, cnt_ref, x_ref, wl_ref, wr_ref, b_ref,
                   o_ref):
    agg = acc_ref[0] + acc_ref[1]
    cnt = cnt_ref[0, :, :1] + cnt_ref[1, :, :1]
    a = agg / jnp.maximum(cnt, 1.0)
    h = _dot_t(a, wl_ref[...]) + _dot_t(x_ref[...], wr_ref[...]) + b_ref[...]
    if normalize:
        nrm = jnp.sqrt(jnp.sum(h * h, axis=1, keepdims=True))
        h = h / jnp.maximum(nrm, 1e-12)
    o_ref[...] = jax.nn.sigmoid(h)


def _make_tc_layer(normalize):
    return pl.pallas_call(
        functools.partial(_tc_layer_body, normalize),
        grid=(N // BLK2,),
        in_specs=[
            pl.BlockSpec((NC, BLK2, D), lambda i: (0, i, 0)),
            pl.BlockSpec((NC, BLK2, D), lambda i: (0, i, 0)),
            pl.BlockSpec((BLK2, D), lambda i: (i, 0)),
            pl.BlockSpec((D, D), lambda i: (0, 0)),
            pl.BlockSpec((D, D), lambda i: (0, 0)),
            pl.BlockSpec((1, D), lambda i: (0, 0)),
        ],
        out_specs=pl.BlockSpec((BLK2, D), lambda i: (i, 0)),
        out_shape=jax.ShapeDtypeStruct((N, D), jnp.float32),
    )


_tc_layer1 = _make_tc_layer(False)
_tc_layer2 = _make_tc_layer(True)


@jax.jit
def kernel(node_feat, edge_index, W1l, b1, W1r, W2l, b2, W2r):
    pad = EPAD - edge_index.shape[1]
    arange_pad = jnp.arange(pad, dtype=jnp.int32) % NT
    src_p = jnp.concatenate([edge_index[0], arange_pad]).reshape(
        NC, NT, KG, G, CH)
    dst_p = jnp.concatenate([edge_index[1], N + arange_pad]).reshape(
        NC, NT, KG, G, CH)
    xtra = jnp.broadcast_to(
        (jnp.arange(2 * CH, dtype=jnp.int32) % NT).reshape(1, 1, 1, 2, CH),
        (NC, NT, KG, 2, CH))
    src_p = jnp.concatenate([src_p, xtra], axis=3)

    zrow = jnp.zeros((ZCH, D), jnp.float32)
    ones = jnp.ones((ZCH, D), jnp.float32)

    cnt, acc1 = _sc_cnt_agg(node_feat, src_p, dst_p, zrow, ones)
    x1 = _tc_layer1(acc1, cnt, node_feat, W1l, W1r, b1.reshape(1, D))
    acc2 = _sc_agg(x1, src_p, dst_p, zrow)
    x2 = _tc_layer2(acc2, cnt, x1, W2l, W2r, b2.reshape(1, D))
    return x2


# speedup vs baseline: 1.1899x; 1.0121x over previous
import functools

import jax
import jax.numpy as jnp
from jax import lax
from jax.experimental import pallas as pl
from jax.experimental.pallas import tpu as pltpu
from jax.experimental.pallas import tpu_sc as plsc

N = 10000
D = 128
NC = 2
NT = 16
CH = 64
K = 160
KG = 2
G = K // KG
EPAD = NC * NT * K * CH
NPAD = 10240
RPT = NPAD // NT
ZCH = 64
BLK = 640
BLK2 = 2000


def _mesh():
    return plsc.VectorSubcoreMesh(core_axis_name="c", subcore_axis_name="s")


def _zero_acc(zrow, buf, acc_s, base):
    pltpu.sync_copy(zrow, buf)
    for t in range(RPT // ZCH):
        pltpu.sync_copy(buf, acc_s.at[pl.ds(base + t * ZCH, ZCH)])


def _stage_out(acc_s, buf, out_hbm, cid, base):
    for t in range(RPT // ZCH):
        sl = pl.ds(base + t * ZCH, ZCH)
        pltpu.sync_copy(acc_s.at[sl], buf)
        pltpu.sync_copy(buf, out_hbm.at[cid, sl])


def _agg_loop(x_hbm, src_hbm, dst_hbm, cid, sid,
              src_v, dst_v, bufa, bufb, acc_s, semga, semgb):
    def group(g, carry):
        pltpu.sync_copy(src_hbm.at[cid, sid, g], src_v)
        pltpu.sync_copy(dst_hbm.at[cid, sid, g], dst_v)
        pltpu.async_copy(x_hbm.at[src_v.at[0]], bufa, semga)

        def pair(i, c2):
            j = 2 * i
            pltpu.async_copy(x_hbm.at[src_v.at[j + 1]], bufb, semgb)
            pltpu.make_async_copy(x_hbm.at[src_v.at[j]], bufa, semga).wait()
            pltpu.sync_copy(bufa, acc_s.at[dst_v.at[j]], add=True)
            pltpu.async_copy(x_hbm.at[src_v.at[j + 2]], bufa, semga)
            pltpu.make_async_copy(x_hbm.at[src_v.at[j + 1]], bufb,
                                  semgb).wait()
            pltpu.sync_copy(bufb, acc_s.at[dst_v.at[j + 1]], add=True)
            return c2

        lax.fori_loop(0, G // 2, pair, 0)
        pltpu.make_async_copy(x_hbm.at[src_v.at[G]], bufa, semga).wait()
        return carry

    lax.fori_loop(0, KG, group, 0)


def _count_loop(dst_hbm, cid, sid, dst_v, buf, acc_s):
    def group(g, carry):
        pltpu.sync_copy(dst_hbm.at[cid, sid, g], dst_v)

        def step(j, c2):
            pltpu.sync_copy(buf, acc_s.at[dst_v.at[j]], add=True)
            return c2

        lax.fori_loop(0, G, step, 0)
        return carry

    lax.fori_loop(0, KG, group, 0)


def _sc_cnt_agg_body(x_hbm, src_hbm, dst_hbm, zrow, ones,
                     cnt_out, acc_out,
                     src_v, dst_v, bufa, bufb, acc_s,
                     semga, semgb):
    cid = lax.axis_index("c")
    sid = lax.axis_index("s")
    base = sid * RPT

    _zero_acc(zrow, bufa, acc_s, base)
    pltpu.sync_copy(ones, bufa)
    plsc.subcore_barrier()
    _count_loop(dst_hbm, cid, sid, dst_v, bufa, acc_s)
    plsc.subcore_barrier()
    _stage_out(acc_s, bufa, cnt_out, cid, base)

    _zero_acc(zrow, bufa, acc_s, base)
    plsc.subcore_barrier()
    _agg_loop(x_hbm, src_hbm, dst_hbm, cid, sid,
              src_v, dst_v, bufa, bufb, acc_s, semga, semgb)
    plsc.subcore_barrier()
    _stage_out(acc_s, bufa, acc_out, cid, base)


_sc_cnt_agg = pl.kernel(
    _sc_cnt_agg_body,
    out_type=[jax.ShapeDtypeStruct((NC, NPAD, D), jnp.float32),
              jax.ShapeDtypeStruct((NC, NPAD, D), jnp.float32)],
    mesh=_mesh(),
    scratch_types=[
        pltpu.VMEM((G + 2, CH), jnp.int32),
        pltpu.VMEM((G, CH), jnp.int32),
        pltpu.VMEM((CH, D), jnp.float32),
        pltpu.VMEM((CH, D), jnp.float32),
        pltpu.VMEM_SHARED((NPAD, D), jnp.float32),
        pltpu.SemaphoreType.DMA,
        pltpu.SemaphoreType.DMA,
    ],
)


def _sc_agg_body(x_hbm, src_hbm, dst_hbm, zrow, acc_out,
                 src_v, dst_v, bufa, bufb, acc_s,
                 semga, semgb):
    cid = lax.axis_index("c")
    sid = lax.axis_index("s")
    base = sid * RPT

    _zero_acc(zrow, bufa, acc_s, base)
    plsc.subcore_barrier()
    _agg_loop(x_hbm, src_hbm, dst_hbm, cid, sid,
              src_v, dst_v, bufa, bufb, acc_s, semga, semgb)
    plsc.subcore_barrier()
    _stage_out(acc_s, bufa, acc_out, cid, base)


_sc_agg = pl.kernel(
    _sc_agg_body,
    out_type=jax.ShapeDtypeStruct((NC, NPAD, D), jnp.float32),
    mesh=_mesh(),
    scratch_types=[
        pltpu.VMEM((G + 2, CH), jnp.int32),
        pltpu.VMEM((G, CH), jnp.int32),
        pltpu.VMEM((CH, D), jnp.float32),
        pltpu.VMEM((CH, D), jnp.float32),
        pltpu.VMEM_SHARED((NPAD, D), jnp.float32),
        pltpu.SemaphoreType.DMA,
        pltpu.SemaphoreType.DMA,
    ],
)


def _dot_t(a, w):
    return lax.dot_general(a, w, (((1,), (1,)), ((), ())),
                           preferred_element_type=jnp.float32)


def _tc_layer_body(normalize, acc_ref, cnt_ref, x_ref, wl_ref, wr_ref, b_ref,
                   o_ref):
    agg = acc_ref[0] + acc_ref[1]
    cnt = cnt_ref[0, :, :1] + cnt_ref[1, :, :1]
    a = agg / jnp.maximum(cnt, 1.0)
    h = _dot_t(a, wl_ref[...]) + _dot_t(x_ref[...], wr_ref[...]) + b_ref[...]
    if normalize:
        nrm = jnp.sqrt(jnp.sum(h * h, axis=1, keepdims=True))
        h = h / jnp.maximum(nrm, 1e-12)
    o_ref[...] = jax.nn.sigmoid(h)


def _make_tc_layer(normalize):
    return pl.pallas_call(
        functools.partial(_tc_layer_body, normalize),
        grid=(N // BLK2,),
        in_specs=[
            pl.BlockSpec((NC, BLK2, D), lambda i: (0, i, 0)),
            pl.BlockSpec((NC, BLK2, D), lambda i: (0, i, 0)),
            pl.BlockSpec((BLK2, D), lambda i: (i, 0)),
            pl.BlockSpec((D, D), lambda i: (0, 0)),
            pl.BlockSpec((D, D), lambda i: (0, 0)),
            pl.BlockSpec((1, D), lambda i: (0, 0)),
        ],
        out_specs=pl.BlockSpec((BLK2, D), lambda i: (i, 0)),
        out_shape=jax.ShapeDtypeStruct((N, D), jnp.float32),
    )


_tc_layer1 = _make_tc_layer(False)
_tc_layer2 = _make_tc_layer(True)


@jax.jit
def kernel(node_feat, edge_index, W1l, b1, W1r, W2l, b2, W2r):
    pad = EPAD - edge_index.shape[1]
    arange_pad = jnp.arange(pad, dtype=jnp.int32) % NT
    src_p = jnp.concatenate([edge_index[0], arange_pad]).reshape(
        NC, NT, KG, G, CH)
    dst_p = jnp.concatenate([edge_index[1], N + arange_pad]).reshape(
        NC, NT, KG, G, CH)
    xtra = jnp.broadcast_to(
        (jnp.arange(2 * CH, dtype=jnp.int32) % NT).reshape(1, 1, 1, 2, CH),
        (NC, NT, KG, 2, CH))
    src_p = jnp.concatenate([src_p, xtra], axis=3)

    zrow = jnp.zeros((ZCH, D), jnp.float32)
    ones = jnp.ones((ZCH, D), jnp.float32)

    cnt, acc1 = _sc_cnt_agg(node_feat, src_p, dst_p, zrow, ones)
    x1 = _tc_layer1(acc1, cnt, node_feat, W1l, W1r, b1.reshape(1, D))
    acc2 = _sc_agg(x1, src_p, dst_p, zrow)
    x2 = _tc_layer2(acc2, cnt, x1, W2l, W2r, b2.reshape(1, D))
    return x2


# speedup vs baseline: 1.1930x; 1.0026x over previous
import functools

import jax
import jax.numpy as jnp
from jax import lax
from jax.experimental import pallas as pl
from jax.experimental.pallas import tpu as pltpu
from jax.experimental.pallas import tpu_sc as plsc

N = 10000
D = 128
NC = 2
NT = 16
CH = 64
K = 160
KG = 2
G = K // KG
EPAD = NC * NT * K * CH
NPAD = 10240
RPT = NPAD // NT
ZCH = 64
BLK = 640
BLK2 = 5000


def _mesh():
    return plsc.VectorSubcoreMesh(core_axis_name="c", subcore_axis_name="s")


def _zero_acc(zrow, buf, acc_s, base):
    pltpu.sync_copy(zrow, buf)
    for t in range(RPT // ZCH):
        pltpu.sync_copy(buf, acc_s.at[pl.ds(base + t * ZCH, ZCH)])


def _stage_out(acc_s, buf, out_hbm, cid, base):
    for t in range(RPT // ZCH):
        sl = pl.ds(base + t * ZCH, ZCH)
        pltpu.sync_copy(acc_s.at[sl], buf)
        pltpu.sync_copy(buf, out_hbm.at[cid, sl])


def _agg_loop(x_hbm, src_hbm, dst_hbm, cid, sid,
              src_v, dst_v, bufa, bufb, acc_s, semga, semgb):
    def group(g, carry):
        pltpu.sync_copy(src_hbm.at[cid, sid, g], src_v)
        pltpu.sync_copy(dst_hbm.at[cid, sid, g], dst_v)
        pltpu.async_copy(x_hbm.at[src_v.at[0]], bufa, semga)

        def pair(i, c2):
            j = 2 * i
            pltpu.async_copy(x_hbm.at[src_v.at[j + 1]], bufb, semgb)
            pltpu.make_async_copy(x_hbm.at[src_v.at[j]], bufa, semga).wait()
            pltpu.sync_copy(bufa, acc_s.at[dst_v.at[j]], add=True)
            pltpu.async_copy(x_hbm.at[src_v.at[j + 2]], bufa, semga)
            pltpu.make_async_copy(x_hbm.at[src_v.at[j + 1]], bufb,
                                  semgb).wait()
            pltpu.sync_copy(bufb, acc_s.at[dst_v.at[j + 1]], add=True)
            return c2

        lax.fori_loop(0, G // 2, pair, 0)
        pltpu.make_async_copy(x_hbm.at[src_v.at[G]], bufa, semga).wait()
        return carry

    lax.fori_loop(0, KG, group, 0)


def _count_loop(dst_hbm, cid, sid, dst_v, buf, acc_s):
    def group(g, carry):
        pltpu.sync_copy(dst_hbm.at[cid, sid, g], dst_v)

        def step(j, c2):
            pltpu.sync_copy(buf, acc_s.at[dst_v.at[j]], add=True)
            return c2

        lax.fori_loop(0, G, step, 0)
        return carry

    lax.fori_loop(0, KG, group, 0)


def _sc_cnt_agg_body(x_hbm, src_hbm, dst_hbm, zrow, ones,
                     cnt_out, acc_out,
                     src_v, dst_v, bufa, bufb, acc_s,
                     semga, semgb):
    cid = lax.axis_index("c")
    sid = lax.axis_index("s")
    base = sid * RPT

    _zero_acc(zrow, bufa, acc_s, base)
    pltpu.sync_copy(ones, bufa)
    plsc.subcore_barrier()
    _count_loop(dst_hbm, cid, sid, dst_v, bufa, acc_s)
    plsc.subcore_barrier()
    _stage_out(acc_s, bufa, cnt_out, cid, base)

    _zero_acc(zrow, bufa, acc_s, base)
    plsc.subcore_barrier()
    _agg_loop(x_hbm, src_hbm, dst_hbm, cid, sid,
              src_v, dst_v, bufa, bufb, acc_s, semga, semgb)
    plsc.subcore_barrier()
    _stage_out(acc_s, bufa, acc_out, cid, base)


_sc_cnt_agg = pl.kernel(
    _sc_cnt_agg_body,
    out_type=[jax.ShapeDtypeStruct((NC, NPAD, D), jnp.float32),
              jax.ShapeDtypeStruct((NC, NPAD, D), jnp.float32)],
    mesh=_mesh(),
    scratch_types=[
        pltpu.VMEM((G + 2, CH), jnp.int32),
        pltpu.VMEM((G, CH), jnp.int32),
        pltpu.VMEM((CH, D), jnp.float32),
        pltpu.VMEM((CH, D), jnp.float32),
        pltpu.VMEM_SHARED((NPAD, D), jnp.float32),
        pltpu.SemaphoreType.DMA,
        pltpu.SemaphoreType.DMA,
    ],
)


def _sc_agg_body(x_hbm, src_hbm, dst_hbm, zrow, acc_out,
                 src_v, dst_v, bufa, bufb, acc_s,
                 semga, semgb):
    cid = lax.axis_index("c")
    sid = lax.axis_index("s")
    base = sid * RPT

    _zero_acc(zrow, bufa, acc_s, base)
    plsc.subcore_barrier()
    _agg_loop(x_hbm, src_hbm, dst_hbm, cid, sid,
              src_v, dst_v, bufa, bufb, acc_s, semga, semgb)
    plsc.subcore_barrier()
    _stage_out(acc_s, bufa, acc_out, cid, base)


_sc_agg = pl.kernel(
    _sc_agg_body,
    out_type=jax.ShapeDtypeStruct((NC, NPAD, D), jnp.float32),
    mesh=_mesh(),
    scratch_types=[
        pltpu.VMEM((G + 2, CH), jnp.int32),
        pltpu.VMEM((G, CH), jnp.int32),
        pltpu.VMEM((CH, D), jnp.float32),
        pltpu.VMEM((CH, D), jnp.float32),
        pltpu.VMEM_SHARED((NPAD, D), jnp.float32),
        pltpu.SemaphoreType.DMA,
        pltpu.SemaphoreType.DMA,
    ],
)


def _dot_t(a, w):
    return lax.dot_general(a, w, (((1,), (1,)), ((), ())),
                           preferred_element_type=jnp.float32)


def _tc_layer_body(normalize, acc_ref, cnt_ref, x_ref, wl_ref, wr_ref, b_ref,
                   o_ref):
    agg = acc_ref[0] + acc_ref[1]
    cnt = cnt_ref[0, :, :1] + cnt_ref[1, :, :1]
    a = agg / jnp.maximum(cnt, 1.0)
    h = _dot_t(a, wl_ref[...]) + _dot_t(x_ref[...], wr_ref[...]) + b_ref[...]
    if normalize:
        nrm = jnp.sqrt(jnp.sum(h * h, axis=1, keepdims=True))
        h = h / jnp.maximum(nrm, 1e-12)
    o_ref[...] = jax.nn.sigmoid(h)


def _make_tc_layer(normalize):
    return pl.pallas_call(
        functools.partial(_tc_layer_body, normalize),
        grid=(N // BLK2,),
        in_specs=[
            pl.BlockSpec((NC, BLK2, D), lambda i: (0, i, 0)),
            pl.BlockSpec((NC, BLK2, D), lambda i: (0, i, 0)),
            pl.BlockSpec((BLK2, D), lambda i: (i, 0)),
            pl.BlockSpec((D, D), lambda i: (0, 0)),
            pl.BlockSpec((D, D), lambda i: (0, 0)),
            pl.BlockSpec((1, D), lambda i: (0, 0)),
        ],
        out_specs=pl.BlockSpec((BLK2, D), lambda i: (i, 0)),
        out_shape=jax.ShapeDtypeStruct((N, D), jnp.float32),
    )


_tc_layer1 = _make_tc_layer(False)
_tc_layer2 = _make_tc_layer(True)


@jax.jit
def kernel(node_feat, edge_index, W1l, b1, W1r, W2l, b2, W2r):
    pad = EPAD - edge_index.shape[1]
    arange_pad = jnp.arange(pad, dtype=jnp.int32) % NT
    src_p = jnp.concatenate([edge_index[0], arange_pad]).reshape(
        NC, NT, KG, G, CH)
    dst_p = jnp.concatenate([edge_index[1], N + arange_pad]).reshape(
        NC, NT, KG, G, CH)
    xtra = jnp.broadcast_to(
        (jnp.arange(2 * CH, dtype=jnp.int32) % NT).reshape(1, 1, 1, 2, CH),
        (NC, NT, KG, 2, CH))
    src_p = jnp.concatenate([src_p, xtra], axis=3)

    zrow = jnp.zeros((ZCH, D), jnp.float32)
    ones = jnp.ones((ZCH, D), jnp.float32)

    cnt, acc1 = _sc_cnt_agg(node_feat, src_p, dst_p, zrow, ones)
    x1 = _tc_layer1(acc1, cnt, node_feat, W1l, W1r, b1.reshape(1, D))
    acc2 = _sc_agg(x1, src_p, dst_p, zrow)
    x2 = _tc_layer2(acc2, cnt, x1, W2l, W2r, b2.reshape(1, D))
    return x2
